# baseline (speedup 1.0000x reference)
import functools

import jax
import jax.numpy as jnp
from jax import lax
from jax.experimental import pallas as pl
from jax.experimental.pallas import tpu as pltpu
from jax.experimental.pallas import tpu_sc as plsc

N = 10000
E = 320000
NB = 20
NBP = 24
CUTOFF = 5.0
NUM_EL = 120
F = 128

NC = 2
NS = 16
NW = NC * NS
EPW = E // NW
NPIECE = 2
EP = E // NPIECE
EPWS = EP // NW
NPSA = 624
NTAIL = N - NS * NPSA
STG = 48
NSTG = NPSA // STG
CHUNK = 40
NCHUNK = EPWS // CHUNK
RING = 5

BE = 2000
NBLK = EP // BE

BN = 1000
NNBLK = N // BN


def _sigmoid(x):
    return 1.0 / (1.0 + jnp.exp(-x))


def _silu(x):
    return x * _sigmoid(x)


def _silu_t(x):
    return 0.5 * x * (1.0 + jnp.tanh(0.5 * x))


def _precompute_body(emb_ref, sW1_ref, sb1_ref, sW2_ref, sb2_ref,
                     tW1_ref, tb1_ref, tW2_ref, tb2_ref,
                     dW2_ref, db2_ref, eW1_ref, eb1_ref,
                     S2_ref, T2_ref, Wd_ref, cvec_ref):
    emb = emb_ref[...]
    S = _silu(jnp.dot(emb, sW1_ref[...], preferred_element_type=jnp.float32)
              + sb1_ref[...])
    S = jnp.dot(S, sW2_ref[...], preferred_element_type=jnp.float32) + sb2_ref[...]
    T = _silu(jnp.dot(emb, tW1_ref[...], preferred_element_type=jnp.float32)
              + tb1_ref[...])
    T = jnp.dot(T, tW2_ref[...], preferred_element_type=jnp.float32) + tb2_ref[...]
    eW1_d = eW1_ref[0:128, :]
    eW1_s = eW1_ref[128:256, :]
    eW1_t = eW1_ref[256:384, :]
    S2_ref[...] = jnp.dot(S, eW1_s, preferred_element_type=jnp.float32)
    T2_ref[...] = jnp.dot(T, eW1_t, preferred_element_type=jnp.float32)
    Wd_ref[...] = jnp.dot(dW2_ref[...], eW1_d, preferred_element_type=jnp.float32)
    cvec_ref[...] = jnp.dot(db2_ref[...], eW1_d,
                            preferred_element_type=jnp.float32) + eb1_ref[...]


def _precompute(emb, sW1, sb1, sW2, sb2, tW1, tb1, tW2, tb2, dW2, db2, eW1, eb1):
    full = lambda shape: pl.BlockSpec(shape, lambda: tuple(0 for _ in shape))
    return pl.pallas_call(
        _precompute_body,
        grid=(),
        in_specs=[full((NUM_EL, F)), full((F, F)), full((1, F)), full((F, F)),
                  full((1, F)), full((F, F)), full((1, F)), full((F, F)),
                  full((1, F)), full((F, F)), full((1, F)), full((3 * F, F)),
                  full((1, F))],
        out_specs=[full((NUM_EL, F)), full((NUM_EL, F)), full((F, F)),
                   full((1, F))],
        out_shape=[jax.ShapeDtypeStruct((NUM_EL, F), jnp.float32),
                   jax.ShapeDtypeStruct((NUM_EL, F), jnp.float32),
                   jax.ShapeDtypeStruct((F, F), jnp.float32),
                   jax.ShapeDtypeStruct((1, F), jnp.float32)],
    )(emb, sW1, sb1, sW2, sb2, tW1, tb1, tW2, tb2, dW2, db2, eW1, eb1)


def _zgather_body(z_hbm, row_hbm, col_hbm, zr_hbm, zc_hbm,
                  z_v, row_v, col_v, zr_v, zc_v):
    c = lax.axis_index("c")
    s = lax.axis_index("s")
    wid = s * NC + c
    base = wid * EPW
    pltpu.sync_copy(z_hbm, z_v)
    pltpu.sync_copy(row_hbm.at[pl.ds(base, EPW)], row_v)
    pltpu.sync_copy(col_hbm.at[pl.ds(base, EPW)], col_v)

    def body(i, carry):
        off = i * 16
        idx_r = row_v[pl.ds(off, 16)]
        zr_v[pl.ds(off, 16)] = plsc.load_gather(z_v, [idx_r])
        idx_c = col_v[pl.ds(off, 16)]
        zc_v[pl.ds(off, 16)] = plsc.load_gather(z_v, [idx_c])
        return carry

    lax.fori_loop(0, EPW // 16, body, 0)
    pltpu.sync_copy(zr_v, zr_hbm.at[pl.ds(base, EPW)])
    pltpu.sync_copy(zc_v, zc_hbm.at[pl.ds(base, EPW)])


def _zgather(z, row, col):
    mesh = plsc.VectorSubcoreMesh(core_axis_name="c", subcore_axis_name="s",
                                  num_cores=NC, num_subcores=NS)
    k = functools.partial(
        pl.kernel,
        mesh=mesh,
        out_type=[jax.ShapeDtypeStruct((E,), jnp.int32),
                  jax.ShapeDtypeStruct((E,), jnp.int32)],
        scratch_types=[
            pltpu.VMEM((N,), jnp.int32),
            pltpu.VMEM((EPW,), jnp.int32),
            pltpu.VMEM((EPW,), jnp.int32),
            pltpu.VMEM((EPW,), jnp.int32),
            pltpu.VMEM((EPW,), jnp.int32),
        ],
        compiler_params=pltpu.CompilerParams(needs_layout_passes=False),
    )(_zgather_body)
    return k(z, row, col)


def _edge_body(r_ref, zr_ref, zc_ref, dW1_ref, db1_ref, Wd_ref, S2_ref,
               T2_ref, cvec_ref, eW2_ref, eb2_ref, out_ref):
    bf16 = jnp.bfloat16
    rb = r_ref[0]
    zr = zr_ref[0]
    zc = zc_ref[0]
    theta = (jnp.pi / CUTOFF) * rb
    c2 = 2.0 * jnp.cos(theta)
    t1 = jnp.sin(theta) * (jnp.sqrt(2.0 / CUTOFF) / rb)
    rows = [t1]
    tp2, tp1 = jnp.zeros_like(t1), t1
    for _ in range(NB - 1):
        t = c2 * tp1 - tp2
        tp2, tp1 = tp1, t
        rows.append(t)
    rows.extend([jnp.zeros_like(t1)] * (NBP - NB))
    basis = jnp.concatenate(rows, axis=0)
    tdot = (((0,), (0,)), ((), ()))
    hid = _silu_t(lax.dot_general(basis.astype(bf16), dW1_ref[...], tdot,
                                  preferred_element_type=jnp.float32)
                  + db1_ref[...])
    lanes = lax.broadcasted_iota(jnp.int32, (NUM_EL, BE), 0)
    ohr = (lanes == zr).astype(bf16)
    ohc = (lanes == zc).astype(bf16)
    pre = (jnp.dot(hid.astype(bf16), Wd_ref[...],
                   preferred_element_type=jnp.float32)
           + lax.dot_general(ohr, S2_ref[...], tdot,
                             preferred_element_type=jnp.float32)
           + lax.dot_general(ohc, T2_ref[...], tdot,
                             preferred_element_type=jnp.float32)
           + cvec_ref[...])
    out_ref[...] = (jnp.dot(_silu_t(pre).astype(bf16), eW2_ref[...],
                            preferred_element_type=jnp.float32)
                    + eb2_ref[...])


def _edge_feats(r3, zr3, zc3, dW1p, db1, Wd, S2, T2, cvec, eW2, eb2):
    full = lambda shape: pl.BlockSpec(shape, lambda i: tuple(0 for _ in shape))
    return pl.pallas_call(
        _edge_body,
        grid=(NBLK,),
        in_specs=[
            pl.BlockSpec((1, 1, BE), lambda i: (i, 0, 0)),
            pl.BlockSpec((1, 1, BE), lambda i: (i, 0, 0)),
            pl.BlockSpec((1, 1, BE), lambda i: (i, 0, 0)),
            full((NBP, F)), full((1, F)), full((F, F)), full((NUM_EL, F)),
            full((NUM_EL, F)), full((1, F)), full((F, F)), full((1, F)),
        ],
        out_specs=pl.BlockSpec((BE, F), lambda i: (i, 0)),
        out_shape=jax.ShapeDtypeStruct((E, F), jnp.float32),
    )(r3, zr3, zc3, dW1p, db1, Wd, S2, T2, cvec, eW2, eb2)


def _fill_iota(idx_ref, off):
    for k in range(STG // 16):
        idx_ref[pl.ds(k * 16, 16)] = lax.iota(jnp.int32, 16) + (off + k * 16)


def _scatter_body(ef_hbm, row_hbm, zacc_hbm, acc_hbm,
                  idx_v, rows_v, accst_v, iidx_v, acc_sh, *sems):
    lsems = sems[:RING]
    asems = sems[RING:]
    c = lax.axis_index("c")
    s = lax.axis_index("s")
    wid = s * NC + c
    base = wid * EPWS
    pltpu.sync_copy(zacc_hbm, accst_v)

    def initb(j, carry):
        _fill_iota(iidx_v, s * NPSA + j * STG)
        pltpu.sync_copy(accst_v, acc_sh.at[iidx_v])
        return carry

    lax.fori_loop(0, NSTG, initb, 0)

    @pl.when(s == NS - 1)
    def _():
        _fill_iota(iidx_v, N - STG)
        pltpu.sync_copy(accst_v, acc_sh.at[iidx_v])

    plsc.subcore_barrier()

    def _issue_loads(b, g):
        off = base + g * CHUNK
        pltpu.async_copy(row_hbm.at[pl.ds(off, CHUNK)], idx_v.at[b],
                         lsems[b])
        pltpu.async_copy(ef_hbm.at[pl.ds(off, CHUNK)], rows_v.at[b],
                         lsems[b])

    def _wait_loads(b, g):
        off = base + g * CHUNK
        pltpu.make_async_copy(row_hbm.at[pl.ds(off, CHUNK)], idx_v.at[b],
                              lsems[b]).wait()
        pltpu.make_async_copy(ef_hbm.at[pl.ds(off, CHUNK)], rows_v.at[b],
                              lsems[b]).wait()

    def _add_desc(b):
        return pltpu.make_async_copy(rows_v.at[b], acc_sh.at[idx_v.at[b]],
                                     asems[b])

    for b in range(RING):
        _issue_loads(b, b)

    def outer(o, carry):
        for b in range(RING):
            v = o * RING + b
            _wait_loads(b, v)
            _add_desc(b).start(add=True)
            bu = (b + 1) % RING
            u = v - (RING - 1)
            if b == RING - 1:
                @pl.when(o < NCHUNK // RING - 1)
                def _():
                    _add_desc(bu).wait()
                    _issue_loads(bu, u + RING)
            else:
                @pl.when(o > 0)
                def _():
                    _add_desc(bu).wait()
                    _issue_loads(bu, u + RING)
        return carry

    lax.fori_loop(0, NCHUNK // RING, outer, 0)
    for b in range(RING):
        _add_desc(b).wait()
    plsc.subcore_barrier()

    def outb(j, carry):
        off = s * NPSA + j * STG
        _fill_iota(iidx_v, off)
        pltpu.sync_copy(acc_sh.at[iidx_v], accst_v)
        pltpu.sync_copy(accst_v, acc_hbm.at[pl.ds(c * N + off, STG)])
        return carry

    lax.fori_loop(0, NSTG, outb, 0)

    @pl.when(s == NS - 1)
    def _():
        _fill_iota(iidx_v, N - STG)
        pltpu.sync_copy(acc_sh.at[iidx_v], accst_v)
        pltpu.sync_copy(accst_v, acc_hbm.at[pl.ds(c * N + N - STG, STG)])


def _scatter(ef, row, zacc):
    mesh = plsc.VectorSubcoreMesh(core_axis_name="c", subcore_axis_name="s",
                                  num_cores=NC, num_subcores=NS)
    k = functools.partial(
        pl.kernel,
        mesh=mesh,
        out_type=jax.ShapeDtypeStruct((NC * N, F), jnp.float32),
        scratch_types=(
            [pltpu.VMEM((RING, CHUNK), jnp.int32),
             pltpu.VMEM((RING, CHUNK, F), jnp.float32),
             pltpu.VMEM((STG, F), jnp.float32),
             pltpu.VMEM((STG,), jnp.int32),
             pltpu.VMEM_SHARED((N, F), jnp.float32)]
            + [pltpu.SemaphoreType.DMA] * (2 * RING)
        ),
    )(_scatter_body)
    return k(ef, row, zacc)


def _final_body(a_ref, b_ref, c_ref, d_ref, out_ref):
    out_ref[...] = (a_ref[...] + b_ref[...]) + (c_ref[...] + d_ref[...])


def _finalize(acc0, acc1):
    return pl.pallas_call(
        _final_body,
        grid=(NNBLK,),
        in_specs=[
            pl.BlockSpec((BN, F), lambda i: (i, 0)),
            pl.BlockSpec((BN, F), lambda i: (i + NNBLK, 0)),
            pl.BlockSpec((BN, F), lambda i: (i, 0)),
            pl.BlockSpec((BN, F), lambda i: (i + NNBLK, 0)),
        ],
        out_specs=pl.BlockSpec((BN, F), lambda i: (i, 0)),
        out_shape=jax.ShapeDtypeStruct((N, F), jnp.float32),
    )(acc0, acc0, acc1, acc1)


def kernel(z, edge_index, edge_weight, emb, dW1, db1, dW2, db2, sW1, sb1,
           sW2, sb2, tW1, tb1, tW2, tb2, eW1, eb1, eW2, eb2):
    f32 = jnp.float32
    row = edge_index[0].astype(jnp.int32)
    col = edge_index[1].astype(jnp.int32)
    z32 = z.astype(jnp.int32)

    S2, T2, Wd, cvec = _precompute(
        emb.astype(f32), sW1, sb1.reshape(1, F), sW2, sb2.reshape(1, F),
        tW1, tb1.reshape(1, F), tW2, tb2.reshape(1, F),
        dW2, db2.reshape(1, F), eW1, eb1.reshape(1, F))

    zr, zc = _zgather(z32, row, col)

    bf16 = jnp.bfloat16
    dW1p = jnp.pad(dW1, ((0, NBP - NB), (0, 0)))
    zacc = jnp.zeros((STG, F), f32)
    r_pieces = edge_weight.astype(f32).reshape(NPIECE, NBLK, 1, BE)
    zr_pieces = zr.reshape(NPIECE, NBLK, 1, BE)
    zc_pieces = zc.reshape(NPIECE, NBLK, 1, BE)
    row_pieces = row.reshape(NPIECE, EP)

    accs = []
    for p in range(NPIECE):
        ef = _edge_feats(
            r_pieces[p], zr_pieces[p], zc_pieces[p],
            dW1p.astype(bf16), db1.reshape(1, F), Wd.astype(bf16),
            S2.astype(bf16), T2.astype(bf16), cvec, eW2.astype(bf16),
            eb2.reshape(1, F))
        accs.append(_scatter(ef, row_pieces[p], zacc))

    return _finalize(accs[0], accs[1])


# speedup vs baseline: 1.2214x; 1.2214x over previous
import functools

import jax
import jax.numpy as jnp
from jax import lax
from jax.experimental import pallas as pl
from jax.experimental.pallas import tpu as pltpu
from jax.experimental.pallas import tpu_sc as plsc

N = 10000
E = 320000
NB = 20
NBP = 24
CUTOFF = 5.0
NUM_EL = 120
F = 128

NC = 2
NS = 16
NW = NC * NS
EPW = E // NW
NPIECE = 2
EP = E // NPIECE
EPWS = EP // NW
NPSA = 624
NTAIL = N - NS * NPSA
STG = 48
NSTG = NPSA // STG
CHUNK = 40
NCHUNK = EPWS // CHUNK
RING = 5

BE = 3200
NBLK = EP // BE

BN = 1000
NNBLK = N // BN


def _sigmoid(x):
    return 1.0 / (1.0 + jnp.exp(-x))


def _silu(x):
    return x * _sigmoid(x)


def _silu_t(x):
    return 0.5 * x * (1.0 + jnp.tanh(0.5 * x))


def _precompute_body(emb_ref, sW1_ref, sb1_ref, sW2_ref, sb2_ref,
                     tW1_ref, tb1_ref, tW2_ref, tb2_ref,
                     dW2_ref, db2_ref, eW1_ref, eb1_ref,
                     S2_ref, T2_ref, Wd_ref, cvec_ref):
    emb = emb_ref[...]
    S = _silu(jnp.dot(emb, sW1_ref[...], preferred_element_type=jnp.float32)
              + sb1_ref[...])
    S = jnp.dot(S, sW2_ref[...], preferred_element_type=jnp.float32) + sb2_ref[...]
    T = _silu(jnp.dot(emb, tW1_ref[...], preferred_element_type=jnp.float32)
              + tb1_ref[...])
    T = jnp.dot(T, tW2_ref[...], preferred_element_type=jnp.float32) + tb2_ref[...]
    eW1_d = eW1_ref[0:128, :]
    eW1_s = eW1_ref[128:256, :]
    eW1_t = eW1_ref[256:384, :]
    S2_ref[...] = jnp.dot(S, eW1_s, preferred_element_type=jnp.float32)
    T2_ref[...] = jnp.dot(T, eW1_t, preferred_element_type=jnp.float32)
    Wd_ref[...] = jnp.dot(dW2_ref[...], eW1_d, preferred_element_type=jnp.float32)
    cvec_ref[...] = jnp.dot(db2_ref[...], eW1_d,
                            preferred_element_type=jnp.float32) + eb1_ref[...]


def _precompute(emb, sW1, sb1, sW2, sb2, tW1, tb1, tW2, tb2, dW2, db2, eW1, eb1):
    full = lambda shape: pl.BlockSpec(shape, lambda: tuple(0 for _ in shape))
    return pl.pallas_call(
        _precompute_body,
        grid=(),
        in_specs=[full((NUM_EL, F)), full((F, F)), full((1, F)), full((F, F)),
                  full((1, F)), full((F, F)), full((1, F)), full((F, F)),
                  full((1, F)), full((F, F)), full((1, F)), full((3 * F, F)),
                  full((1, F))],
        out_specs=[full((NUM_EL, F)), full((NUM_EL, F)), full((F, F)),
                   full((1, F))],
        out_shape=[jax.ShapeDtypeStruct((NUM_EL, F), jnp.float32),
                   jax.ShapeDtypeStruct((NUM_EL, F), jnp.float32),
                   jax.ShapeDtypeStruct((F, F), jnp.float32),
                   jax.ShapeDtypeStruct((1, F), jnp.float32)],
    )(emb, sW1, sb1, sW2, sb2, tW1, tb1, tW2, tb2, dW2, db2, eW1, eb1)


def _zgather_body(z_hbm, row_hbm, col_hbm, zr_hbm, zc_hbm,
                  z_v, row_v, col_v, zr_v, zc_v):
    c = lax.axis_index("c")
    s = lax.axis_index("s")
    wid = s * NC + c
    base = wid * EPW
    pltpu.sync_copy(z_hbm, z_v)
    pltpu.sync_copy(row_hbm.at[pl.ds(base, EPW)], row_v)
    pltpu.sync_copy(col_hbm.at[pl.ds(base, EPW)], col_v)

    def body(i, carry):
        off = i * 16
        idx_r = row_v[pl.ds(off, 16)]
        zr_v[pl.ds(off, 16)] = plsc.load_gather(z_v, [idx_r])
        idx_c = col_v[pl.ds(off, 16)]
        zc_v[pl.ds(off, 16)] = plsc.load_gather(z_v, [idx_c])
        return carry

    lax.fori_loop(0, EPW // 16, body, 0)
    pltpu.sync_copy(zr_v, zr_hbm.at[pl.ds(base, EPW)])
    pltpu.sync_copy(zc_v, zc_hbm.at[pl.ds(base, EPW)])


def _zgather(z, row, col):
    mesh = plsc.VectorSubcoreMesh(core_axis_name="c", subcore_axis_name="s",
                                  num_cores=NC, num_subcores=NS)
    k = functools.partial(
        pl.kernel,
        mesh=mesh,
        out_type=[jax.ShapeDtypeStruct((E,), jnp.int32),
                  jax.ShapeDtypeStruct((E,), jnp.int32)],
        scratch_types=[
            pltpu.VMEM((N,), jnp.int32),
            pltpu.VMEM((EPW,), jnp.int32),
            pltpu.VMEM((EPW,), jnp.int32),
            pltpu.VMEM((EPW,), jnp.int32),
            pltpu.VMEM((EPW,), jnp.int32),
        ],
        compiler_params=pltpu.CompilerParams(needs_layout_passes=False),
    )(_zgather_body)
    return k(z, row, col)


def _edge_body(r_ref, zr_ref, zc_ref, dW1_ref, db1_ref, Wd_ref, S2_ref,
               T2_ref, cvec_ref, eW2_ref, eb2_ref, out_ref):
    bf16 = jnp.bfloat16
    rb = r_ref[0]
    zr = zr_ref[0]
    zc = zc_ref[0]
    theta = (jnp.pi / CUTOFF) * rb
    c2 = 2.0 * jnp.cos(theta)
    t1 = jnp.sin(theta) * (jnp.sqrt(2.0 / CUTOFF) / rb)
    rows = [t1]
    tp2, tp1 = jnp.zeros_like(t1), t1
    for _ in range(NB - 1):
        t = c2 * tp1 - tp2
        tp2, tp1 = tp1, t
        rows.append(t)
    rows.extend([jnp.zeros_like(t1)] * (NBP - NB))
    basis = jnp.concatenate(rows, axis=0)
    tdot = (((0,), (0,)), ((), ()))
    hid = _silu_t(lax.dot_general(basis.astype(bf16), dW1_ref[...], tdot,
                                  preferred_element_type=jnp.float32)
                  + db1_ref[...])
    lanes = lax.broadcasted_iota(jnp.int32, (NUM_EL, BE), 0)
    ohr = (lanes == zr).astype(bf16)
    ohc = (lanes == zc).astype(bf16)
    pre = (jnp.dot(hid.astype(bf16), Wd_ref[...],
                   preferred_element_type=jnp.float32)
           + lax.dot_general(ohr, S2_ref[...], tdot,
                             preferred_element_type=jnp.float32)
           + lax.dot_general(ohc, T2_ref[...], tdot,
                             preferred_element_type=jnp.float32)
           + cvec_ref[...])
    out_ref[...] = (jnp.dot(_silu_t(pre).astype(bf16), eW2_ref[...],
                            preferred_element_type=jnp.float32)
                    + eb2_ref[...])


def _edge_feats(r3, zr3, zc3, dW1p, db1, Wd, S2, T2, cvec, eW2, eb2):
    full = lambda shape: pl.BlockSpec(shape, lambda i: tuple(0 for _ in shape))
    return pl.pallas_call(
        _edge_body,
        grid=(NBLK,),
        in_specs=[
            pl.BlockSpec((1, 1, BE), lambda i: (i, 0, 0)),
            pl.BlockSpec((1, 1, BE), lambda i: (i, 0, 0)),
            pl.BlockSpec((1, 1, BE), lambda i: (i, 0, 0)),
            full((NBP, F)), full((1, F)), full((F, F)), full((NUM_EL, F)),
            full((NUM_EL, F)), full((1, F)), full((F, F)), full((1, F)),
        ],
        out_specs=pl.BlockSpec((BE, F), lambda i: (i, 0)),
        out_shape=jax.ShapeDtypeStruct((E, F), jnp.float32),
    )(r3, zr3, zc3, dW1p, db1, Wd, S2, T2, cvec, eW2, eb2)


def _fill_iota(idx_ref, off):
    for k in range(STG // 16):
        idx_ref[pl.ds(k * 16, 16)] = lax.iota(jnp.int32, 16) + (off + k * 16)


def _scatter_body(ef_hbm, row_hbm, zacc_hbm, acc_hbm,
                  idx_v, rows_v, accst_v, iidx_v, acc_sh, *sems):
    lsems = sems[:RING]
    asems = sems[RING:]
    c = lax.axis_index("c")
    s = lax.axis_index("s")
    wid = s * NC + c
    base = wid * EPWS
    pltpu.sync_copy(zacc_hbm, accst_v)

    def initb(j, carry):
        _fill_iota(iidx_v, s * NPSA + j * STG)
        pltpu.sync_copy(accst_v, acc_sh.at[iidx_v])
        return carry

    lax.fori_loop(0, NSTG, initb, 0)

    @pl.when(s == NS - 1)
    def _():
        _fill_iota(iidx_v, N - STG)
        pltpu.sync_copy(accst_v, acc_sh.at[iidx_v])

    plsc.subcore_barrier()

    def _issue_loads(b, g):
        off = base + g * CHUNK
        pltpu.async_copy(row_hbm.at[pl.ds(off, CHUNK)], idx_v.at[b],
                         lsems[b])
        pltpu.async_copy(ef_hbm.at[pl.ds(off, CHUNK)], rows_v.at[b],
                         lsems[b])

    def _wait_loads(b, g):
        off = base + g * CHUNK
        pltpu.make_async_copy(row_hbm.at[pl.ds(off, CHUNK)], idx_v.at[b],
                              lsems[b]).wait()
        pltpu.make_async_copy(ef_hbm.at[pl.ds(off, CHUNK)], rows_v.at[b],
                              lsems[b]).wait()

    def _add_desc(b):
        return pltpu.make_async_copy(rows_v.at[b], acc_sh.at[idx_v.at[b]],
                                     asems[b])

    for b in range(RING):
        _issue_loads(b, b)

    def outer(o, carry):
        for b in range(RING):
            v = o * RING + b
            _wait_loads(b, v)
            _add_desc(b).start(add=True)
            bu = (b + 1) % RING
            u = v - (RING - 1)
            if b == RING - 1:
                @pl.when(o < NCHUNK // RING - 1)
                def _():
                    _add_desc(bu).wait()
                    _issue_loads(bu, u + RING)
            else:
                @pl.when(o > 0)
                def _():
                    _add_desc(bu).wait()
                    _issue_loads(bu, u + RING)
        return carry

    lax.fori_loop(0, NCHUNK // RING, outer, 0)
    for b in range(RING):
        _add_desc(b).wait()
    plsc.subcore_barrier()

    def outb(j, carry):
        off = s * NPSA + j * STG
        _fill_iota(iidx_v, off)
        pltpu.sync_copy(acc_sh.at[iidx_v], accst_v)
        pltpu.sync_copy(accst_v, acc_hbm.at[pl.ds(c * N + off, STG)])
        return carry

    lax.fori_loop(0, NSTG, outb, 0)

    @pl.when(s == NS - 1)
    def _():
        _fill_iota(iidx_v, N - STG)
        pltpu.sync_copy(acc_sh.at[iidx_v], accst_v)
        pltpu.sync_copy(accst_v, acc_hbm.at[pl.ds(c * N + N - STG, STG)])


def _scatter(ef, row, zacc):
    mesh = plsc.VectorSubcoreMesh(core_axis_name="c", subcore_axis_name="s",
                                  num_cores=NC, num_subcores=NS)
    k = functools.partial(
        pl.kernel,
        mesh=mesh,
        out_type=jax.ShapeDtypeStruct((NC * N, F), jnp.float32),
        scratch_types=(
            [pltpu.VMEM((RING, CHUNK), jnp.int32),
             pltpu.VMEM((RING, CHUNK, F), jnp.float32),
             pltpu.VMEM((STG, F), jnp.float32),
             pltpu.VMEM((STG,), jnp.int32),
             pltpu.VMEM_SHARED((N, F), jnp.float32)]
            + [pltpu.SemaphoreType.DMA] * (2 * RING)
        ),
    )(_scatter_body)
    return k(ef, row, zacc)


def _final_body(a_ref, b_ref, c_ref, d_ref, out_ref):
    out_ref[...] = (a_ref[...] + b_ref[...]) + (c_ref[...] + d_ref[...])


def _finalize(acc0, acc1):
    return pl.pallas_call(
        _final_body,
        grid=(NNBLK,),
        in_specs=[
            pl.BlockSpec((BN, F), lambda i: (i, 0)),
            pl.BlockSpec((BN, F), lambda i: (i + NNBLK, 0)),
            pl.BlockSpec((BN, F), lambda i: (i, 0)),
            pl.BlockSpec((BN, F), lambda i: (i + NNBLK, 0)),
        ],
        out_specs=pl.BlockSpec((BN, F), lambda i: (i, 0)),
        out_shape=jax.ShapeDtypeStruct((N, F), jnp.float32),
    )(acc0, acc0, acc1, acc1)


def kernel(z, edge_index, edge_weight, emb, dW1, db1, dW2, db2, sW1, sb1,
           sW2, sb2, tW1, tb1, tW2, tb2, eW1, eb1, eW2, eb2):
    f32 = jnp.float32
    row = edge_index[0].astype(jnp.int32)
    col = edge_index[1].astype(jnp.int32)
    z32 = z.astype(jnp.int32)

    S2, T2, Wd, cvec = _precompute(
        emb.astype(f32), sW1, sb1.reshape(1, F), sW2, sb2.reshape(1, F),
        tW1, tb1.reshape(1, F), tW2, tb2.reshape(1, F),
        dW2, db2.reshape(1, F), eW1, eb1.reshape(1, F))

    zr, zc = _zgather(z32, row, col)

    bf16 = jnp.bfloat16
    dW1p = jnp.pad(dW1, ((0, NBP - NB), (0, 0)))
    zacc = jnp.zeros((STG, F), f32)
    r_pieces = edge_weight.astype(f32).reshape(NPIECE, NBLK, 1, BE)
    zr_pieces = zr.reshape(NPIECE, NBLK, 1, BE)
    zc_pieces = zc.reshape(NPIECE, NBLK, 1, BE)
    row_pieces = row.reshape(NPIECE, EP)

    accs = []
    for p in range(NPIECE):
        ef = _edge_feats(
            r_pieces[p], zr_pieces[p], zc_pieces[p],
            dW1p.astype(bf16), db1.reshape(1, F), Wd.astype(bf16),
            S2.astype(bf16), T2.astype(bf16), cvec, eW2.astype(bf16),
            eb2.reshape(1, F))
        accs.append(_scatter(ef, row_pieces[p], zacc))

    return _finalize(accs[0], accs[1])


# speedup vs baseline: 1.5252x; 1.2487x over previous
import functools

import jax
import jax.numpy as jnp
from jax import lax
from jax.experimental import pallas as pl
from jax.experimental.pallas import tpu as pltpu
from jax.experimental.pallas import tpu_sc as plsc

N = 10000
E = 320000
NB = 20
NBP = 24
CUTOFF = 5.0
NUM_EL = 120
F = 128

NC = 2
NS = 16
NW = NC * NS
EPW = E // NW
NPIECE = 2
EP = E // NPIECE
EPWS = EP // NW
NPSA = 624
NTAIL = N - NS * NPSA
STG = 48
NSTG = NPSA // STG
CHUNK = 104
NCHUNK = EPWS // CHUNK
TAILC = EPWS - NCHUNK * CHUNK
RING = 3

BE = 3200
NBLK = EP // BE

BN = 1000
NNBLK = N // BN


def _sigmoid(x):
    return 1.0 / (1.0 + jnp.exp(-x))


def _silu(x):
    return x * _sigmoid(x)


def _silu_t(x):
    return 0.5 * x * (1.0 + jnp.tanh(0.5 * x))


def _precompute_body(emb_ref, sW1_ref, sb1_ref, sW2_ref, sb2_ref,
                     tW1_ref, tb1_ref, tW2_ref, tb2_ref,
                     dW2_ref, db2_ref, eW1_ref, eb1_ref,
                     S2_ref, T2_ref, Wd_ref, cvec_ref):
    emb = emb_ref[...]
    S = _silu(jnp.dot(emb, sW1_ref[...], preferred_element_type=jnp.float32)
              + sb1_ref[...])
    S = jnp.dot(S, sW2_ref[...], preferred_element_type=jnp.float32) + sb2_ref[...]
    T = _silu(jnp.dot(emb, tW1_ref[...], preferred_element_type=jnp.float32)
              + tb1_ref[...])
    T = jnp.dot(T, tW2_ref[...], preferred_element_type=jnp.float32) + tb2_ref[...]
    eW1_d = eW1_ref[0:128, :]
    eW1_s = eW1_ref[128:256, :]
    eW1_t = eW1_ref[256:384, :]
    S2_ref[...] = jnp.dot(S, eW1_s, preferred_element_type=jnp.float32)
    T2_ref[...] = jnp.dot(T, eW1_t, preferred_element_type=jnp.float32)
    Wd_ref[...] = jnp.dot(dW2_ref[...], eW1_d, preferred_element_type=jnp.float32)
    cvec_ref[...] = jnp.dot(db2_ref[...], eW1_d,
                            preferred_element_type=jnp.float32) + eb1_ref[...]


def _precompute(emb, sW1, sb1, sW2, sb2, tW1, tb1, tW2, tb2, dW2, db2, eW1, eb1):
    full = lambda shape: pl.BlockSpec(shape, lambda: tuple(0 for _ in shape))
    return pl.pallas_call(
        _precompute_body,
        grid=(),
        in_specs=[full((NUM_EL, F)), full((F, F)), full((1, F)), full((F, F)),
                  full((1, F)), full((F, F)), full((1, F)), full((F, F)),
                  full((1, F)), full((F, F)), full((1, F)), full((3 * F, F)),
                  full((1, F))],
        out_specs=[full((NUM_EL, F)), full((NUM_EL, F)), full((F, F)),
                   full((1, F))],
        out_shape=[jax.ShapeDtypeStruct((NUM_EL, F), jnp.float32),
                   jax.ShapeDtypeStruct((NUM_EL, F), jnp.float32),
                   jax.ShapeDtypeStruct((F, F), jnp.float32),
                   jax.ShapeDtypeStruct((1, F), jnp.float32)],
    )(emb, sW1, sb1, sW2, sb2, tW1, tb1, tW2, tb2, dW2, db2, eW1, eb1)


def _zgather_body(z_hbm, row_hbm, col_hbm, zr_hbm, zc_hbm,
                  z_v, row_v, col_v, zr_v, zc_v):
    c = lax.axis_index("c")
    s = lax.axis_index("s")
    wid = s * NC + c
    base = wid * EPW
    pltpu.sync_copy(z_hbm, z_v)
    pltpu.sync_copy(row_hbm.at[pl.ds(base, EPW)], row_v)
    pltpu.sync_copy(col_hbm.at[pl.ds(base, EPW)], col_v)

    def body(i, carry):
        off = i * 16
        idx_r = row_v[pl.ds(off, 16)]
        zr_v[pl.ds(off, 16)] = plsc.load_gather(z_v, [idx_r])
        idx_c = col_v[pl.ds(off, 16)]
        zc_v[pl.ds(off, 16)] = plsc.load_gather(z_v, [idx_c])
        return carry

    lax.fori_loop(0, EPW // 16, body, 0)
    pltpu.sync_copy(zr_v, zr_hbm.at[pl.ds(base, EPW)])
    pltpu.sync_copy(zc_v, zc_hbm.at[pl.ds(base, EPW)])


def _zgather(z, row, col):
    mesh = plsc.VectorSubcoreMesh(core_axis_name="c", subcore_axis_name="s",
                                  num_cores=NC, num_subcores=NS)
    k = functools.partial(
        pl.kernel,
        mesh=mesh,
        out_type=[jax.ShapeDtypeStruct((E,), jnp.int32),
                  jax.ShapeDtypeStruct((E,), jnp.int32)],
        scratch_types=[
            pltpu.VMEM((N,), jnp.int32),
            pltpu.VMEM((EPW,), jnp.int32),
            pltpu.VMEM((EPW,), jnp.int32),
            pltpu.VMEM((EPW,), jnp.int32),
            pltpu.VMEM((EPW,), jnp.int32),
        ],
        compiler_params=pltpu.CompilerParams(needs_layout_passes=False),
    )(_zgather_body)
    return k(z, row, col)


def _edge_body(r_ref, zr_ref, zc_ref, dW1_ref, db1_ref, Wd_ref, S2_ref,
               T2_ref, cvec_ref, eW2_ref, eb2_ref, out_ref):
    bf16 = jnp.bfloat16
    rb = r_ref[0]
    zr = zr_ref[0]
    zc = zc_ref[0]
    theta = (jnp.pi / CUTOFF) * rb
    c2 = 2.0 * jnp.cos(theta)
    t1 = jnp.sin(theta) * (jnp.sqrt(2.0 / CUTOFF) / rb)
    rows = [t1]
    tp2, tp1 = jnp.zeros_like(t1), t1
    for _ in range(NB - 1):
        t = c2 * tp1 - tp2
        tp2, tp1 = tp1, t
        rows.append(t)
    rows.extend([jnp.zeros_like(t1)] * (NBP - NB))
    basis = jnp.concatenate(rows, axis=0)
    tdot = (((0,), (0,)), ((), ()))
    hid = _silu_t(lax.dot_general(basis.astype(bf16), dW1_ref[...], tdot,
                                  preferred_element_type=jnp.float32)
                  + db1_ref[...])
    lanes = lax.broadcasted_iota(jnp.int32, (NUM_EL, BE), 0)
    ohr = (lanes == zr).astype(bf16)
    ohc = (lanes == zc).astype(bf16)
    pre = (jnp.dot(hid.astype(bf16), Wd_ref[...],
                   preferred_element_type=jnp.float32)
           + lax.dot_general(ohr, S2_ref[...], tdot,
                             preferred_element_type=jnp.float32)
           + lax.dot_general(ohc, T2_ref[...], tdot,
                             preferred_element_type=jnp.float32)
           + cvec_ref[...])
    out_ref[...] = (jnp.dot(_silu_t(pre).astype(bf16), eW2_ref[...],
                            preferred_element_type=jnp.float32)
                    + eb2_ref[...])


def _edge_feats(r3, zr3, zc3, dW1p, db1, Wd, S2, T2, cvec, eW2, eb2):
    full = lambda shape: pl.BlockSpec(shape, lambda i: tuple(0 for _ in shape))
    return pl.pallas_call(
        _edge_body,
        grid=(NBLK,),
        in_specs=[
            pl.BlockSpec((1, 1, BE), lambda i: (i, 0, 0)),
            pl.BlockSpec((1, 1, BE), lambda i: (i, 0, 0)),
            pl.BlockSpec((1, 1, BE), lambda i: (i, 0, 0)),
            full((NBP, F)), full((1, F)), full((F, F)), full((NUM_EL, F)),
            full((NUM_EL, F)), full((1, F)), full((F, F)), full((1, F)),
        ],
        out_specs=pl.BlockSpec((BE, F), lambda i: (i, 0)),
        out_shape=jax.ShapeDtypeStruct((E, F), jnp.float32),
    )(r3, zr3, zc3, dW1p, db1, Wd, S2, T2, cvec, eW2, eb2)


def _fill_iota(idx_ref, off):
    for k in range(STG // 16):
        idx_ref[pl.ds(k * 16, 16)] = lax.iota(jnp.int32, 16) + (off + k * 16)


def _scatter_body(ef_hbm, row_hbm, zacc_hbm, acc_hbm,
                  idx_v, rows_v, accst_v, iidx_v, tidx_v, trows_v,
                  acc_sh, *sems):
    lsems = sems[:RING]
    asems = sems[RING:]
    c = lax.axis_index("c")
    s = lax.axis_index("s")
    wid = s * NC + c
    base = wid * EPWS
    pltpu.sync_copy(zacc_hbm, accst_v)

    def initb(j, carry):
        _fill_iota(iidx_v, s * NPSA + j * STG)
        pltpu.sync_copy(accst_v, acc_sh.at[iidx_v])
        return carry

    lax.fori_loop(0, NSTG, initb, 0)

    @pl.when(s == NS - 1)
    def _():
        _fill_iota(iidx_v, N - STG)
        pltpu.sync_copy(accst_v, acc_sh.at[iidx_v])

    plsc.subcore_barrier()

    def _issue_loads(b, g):
        off = base + g * CHUNK
        pltpu.async_copy(row_hbm.at[pl.ds(off, CHUNK)], idx_v.at[b],
                         lsems[b])
        pltpu.async_copy(ef_hbm.at[pl.ds(off, CHUNK)], rows_v.at[b],
                         lsems[b])

    def _wait_loads(b, g):
        off = base + g * CHUNK
        pltpu.make_async_copy(row_hbm.at[pl.ds(off, CHUNK)], idx_v.at[b],
                              lsems[b]).wait()
        pltpu.make_async_copy(ef_hbm.at[pl.ds(off, CHUNK)], rows_v.at[b],
                              lsems[b]).wait()

    def _add_desc(b):
        return pltpu.make_async_copy(rows_v.at[b], acc_sh.at[idx_v.at[b]],
                                     asems[b])

    for b in range(RING):
        _issue_loads(b, b)

    def outer(o, carry):
        for b in range(RING):
            v = o * RING + b
            _wait_loads(b, v)
            _add_desc(b).start(add=True)
            bu = (b + 1) % RING
            u = v - (RING - 1)
            if b == RING - 1:
                @pl.when(o < NCHUNK // RING - 1)
                def _():
                    _add_desc(bu).wait()
                    _issue_loads(bu, u + RING)
            else:
                @pl.when(o > 0)
                def _():
                    _add_desc(bu).wait()
                    _issue_loads(bu, u + RING)
        return carry

    lax.fori_loop(0, NCHUNK // RING, outer, 0)
    for b in range(RING):
        _add_desc(b).wait()
    toff = base + NCHUNK * CHUNK
    pltpu.sync_copy(row_hbm.at[pl.ds(toff, TAILC)], tidx_v)
    pltpu.sync_copy(ef_hbm.at[pl.ds(toff, TAILC)], trows_v)
    pltpu.sync_copy(trows_v, acc_sh.at[tidx_v], add=True)
    plsc.subcore_barrier()

    def outb(j, carry):
        off = s * NPSA + j * STG
        _fill_iota(iidx_v, off)
        pltpu.sync_copy(acc_sh.at[iidx_v], accst_v)
        pltpu.sync_copy(accst_v, acc_hbm.at[pl.ds(c * N + off, STG)])
        return carry

    lax.fori_loop(0, NSTG, outb, 0)

    @pl.when(s == NS - 1)
    def _():
        _fill_iota(iidx_v, N - STG)
        pltpu.sync_copy(acc_sh.at[iidx_v], accst_v)
        pltpu.sync_copy(accst_v, acc_hbm.at[pl.ds(c * N + N - STG, STG)])


def _scatter(ef, row, zacc):
    mesh = plsc.VectorSubcoreMesh(core_axis_name="c", subcore_axis_name="s",
                                  num_cores=NC, num_subcores=NS)
    k = functools.partial(
        pl.kernel,
        mesh=mesh,
        out_type=jax.ShapeDtypeStruct((NC * N, F), jnp.float32),
        scratch_types=(
            [pltpu.VMEM((RING, CHUNK), jnp.int32),
             pltpu.VMEM((RING, CHUNK, F), jnp.float32),
             pltpu.VMEM((STG, F), jnp.float32),
             pltpu.VMEM((STG,), jnp.int32),
             pltpu.VMEM((TAILC,), jnp.int32),
             pltpu.VMEM((TAILC, F), jnp.float32),
             pltpu.VMEM_SHARED((N, F), jnp.float32)]
            + [pltpu.SemaphoreType.DMA] * (2 * RING)
        ),
    )(_scatter_body)
    return k(ef, row, zacc)


def _final_body(a_ref, b_ref, c_ref, d_ref, out_ref):
    out_ref[...] = (a_ref[...] + b_ref[...]) + (c_ref[...] + d_ref[...])


def _finalize(acc0, acc1):
    return pl.pallas_call(
        _final_body,
        grid=(NNBLK,),
        in_specs=[
            pl.BlockSpec((BN, F), lambda i: (i, 0)),
            pl.BlockSpec((BN, F), lambda i: (i + NNBLK, 0)),
            pl.BlockSpec((BN, F), lambda i: (i, 0)),
            pl.BlockSpec((BN, F), lambda i: (i + NNBLK, 0)),
        ],
        out_specs=pl.BlockSpec((BN, F), lambda i: (i, 0)),
        out_shape=jax.ShapeDtypeStruct((N, F), jnp.float32),
    )(acc0, acc0, acc1, acc1)


def kernel(z, edge_index, edge_weight, emb, dW1, db1, dW2, db2, sW1, sb1,
           sW2, sb2, tW1, tb1, tW2, tb2, eW1, eb1, eW2, eb2):
    f32 = jnp.float32
    row = edge_index[0].astype(jnp.int32)
    col = edge_index[1].astype(jnp.int32)
    z32 = z.astype(jnp.int32)

    S2, T2, Wd, cvec = _precompute(
        emb.astype(f32), sW1, sb1.reshape(1, F), sW2, sb2.reshape(1, F),
        tW1, tb1.reshape(1, F), tW2, tb2.reshape(1, F),
        dW2, db2.reshape(1, F), eW1, eb1.reshape(1, F))

    zr, zc = _zgather(z32, row, col)

    bf16 = jnp.bfloat16
    dW1p = jnp.pad(dW1, ((0, NBP - NB), (0, 0)))
    zacc = jnp.zeros((STG, F), f32)
    r_pieces = edge_weight.astype(f32).reshape(NPIECE, NBLK, 1, BE)
    zr_pieces = zr.reshape(NPIECE, NBLK, 1, BE)
    zc_pieces = zc.reshape(NPIECE, NBLK, 1, BE)
    row_pieces = row.reshape(NPIECE, EP)

    accs = []
    for p in range(NPIECE):
        ef = _edge_feats(
            r_pieces[p], zr_pieces[p], zc_pieces[p],
            dW1p.astype(bf16), db1.reshape(1, F), Wd.astype(bf16),
            S2.astype(bf16), T2.astype(bf16), cvec, eW2.astype(bf16),
            eb2.reshape(1, F))
        accs.append(_scatter(ef, row_pieces[p], zacc))

    return _finalize(accs[0], accs[1])


# speedup vs baseline: 1.5989x; 1.0483x over previous
import functools

import jax
import jax.numpy as jnp
from jax import lax
from jax.experimental import pallas as pl
from jax.experimental.pallas import tpu as pltpu
from jax.experimental.pallas import tpu_sc as plsc

N = 10000
E = 320000
NB = 20
NBP = 24
CUTOFF = 5.0
NUM_EL = 120
F = 128

NC = 2
NS = 16
NW = NC * NS
EPW = E // NW
NPIECE = 2
EP = E // NPIECE
EPWS = EP // NW
NPSA = 624
NTAIL = N - NS * NPSA
STG = 48
NSTG = NPSA // STG
CHUNK = 104
NCHUNK = EPWS // CHUNK
TAILC = EPWS - NCHUNK * CHUNK
RING = 3

BE = 6400
NBLK = EP // BE

BN = 1000
NNBLK = N // BN


def _sigmoid(x):
    return 1.0 / (1.0 + jnp.exp(-x))


def _silu(x):
    return x * _sigmoid(x)


def _silu_t(x):
    return 0.5 * x * (1.0 + jnp.tanh(0.5 * x))


def _precompute_body(emb_ref, sW1_ref, sb1_ref, sW2_ref, sb2_ref,
                     tW1_ref, tb1_ref, tW2_ref, tb2_ref,
                     dW2_ref, db2_ref, eW1_ref, eb1_ref,
                     S2_ref, T2_ref, Wd_ref, cvec_ref):
    emb = emb_ref[...]
    S = _silu(jnp.dot(emb, sW1_ref[...], preferred_element_type=jnp.float32)
              + sb1_ref[...])
    S = jnp.dot(S, sW2_ref[...], preferred_element_type=jnp.float32) + sb2_ref[...]
    T = _silu(jnp.dot(emb, tW1_ref[...], preferred_element_type=jnp.float32)
              + tb1_ref[...])
    T = jnp.dot(T, tW2_ref[...], preferred_element_type=jnp.float32) + tb2_ref[...]
    eW1_d = eW1_ref[0:128, :]
    eW1_s = eW1_ref[128:256, :]
    eW1_t = eW1_ref[256:384, :]
    S2_ref[...] = jnp.dot(S, eW1_s, preferred_element_type=jnp.float32)
    T2_ref[...] = jnp.dot(T, eW1_t, preferred_element_type=jnp.float32)
    Wd_ref[...] = jnp.dot(dW2_ref[...], eW1_d, preferred_element_type=jnp.float32)
    cvec_ref[...] = jnp.dot(db2_ref[...], eW1_d,
                            preferred_element_type=jnp.float32) + eb1_ref[...]


def _precompute(emb, sW1, sb1, sW2, sb2, tW1, tb1, tW2, tb2, dW2, db2, eW1, eb1):
    full = lambda shape: pl.BlockSpec(shape, lambda: tuple(0 for _ in shape))
    return pl.pallas_call(
        _precompute_body,
        grid=(),
        in_specs=[full((NUM_EL, F)), full((F, F)), full((1, F)), full((F, F)),
                  full((1, F)), full((F, F)), full((1, F)), full((F, F)),
                  full((1, F)), full((F, F)), full((1, F)), full((3 * F, F)),
                  full((1, F))],
        out_specs=[full((NUM_EL, F)), full((NUM_EL, F)), full((F, F)),
                   full((1, F))],
        out_shape=[jax.ShapeDtypeStruct((NUM_EL, F), jnp.float32),
                   jax.ShapeDtypeStruct((NUM_EL, F), jnp.float32),
                   jax.ShapeDtypeStruct((F, F), jnp.float32),
                   jax.ShapeDtypeStruct((1, F), jnp.float32)],
    )(emb, sW1, sb1, sW2, sb2, tW1, tb1, tW2, tb2, dW2, db2, eW1, eb1)


def _zgather_body(z_hbm, row_hbm, col_hbm, zr_hbm, zc_hbm,
                  z_v, row_v, col_v, zr_v, zc_v):
    c = lax.axis_index("c")
    s = lax.axis_index("s")
    wid = s * NC + c
    base = wid * EPW
    pltpu.sync_copy(z_hbm, z_v)
    pltpu.sync_copy(row_hbm.at[pl.ds(base, EPW)], row_v)
    pltpu.sync_copy(col_hbm.at[pl.ds(base, EPW)], col_v)

    def body(i, carry):
        off = i * 16
        idx_r = row_v[pl.ds(off, 16)]
        zr_v[pl.ds(off, 16)] = plsc.load_gather(z_v, [idx_r])
        idx_c = col_v[pl.ds(off, 16)]
        zc_v[pl.ds(off, 16)] = plsc.load_gather(z_v, [idx_c])
        return carry

    lax.fori_loop(0, EPW // 16, body, 0)
    pltpu.sync_copy(zr_v, zr_hbm.at[pl.ds(base, EPW)])
    pltpu.sync_copy(zc_v, zc_hbm.at[pl.ds(base, EPW)])


def _zgather(z, row, col):
    mesh = plsc.VectorSubcoreMesh(core_axis_name="c", subcore_axis_name="s",
                                  num_cores=NC, num_subcores=NS)
    k = functools.partial(
        pl.kernel,
        mesh=mesh,
        out_type=[jax.ShapeDtypeStruct((E,), jnp.int32),
                  jax.ShapeDtypeStruct((E,), jnp.int32)],
        scratch_types=[
            pltpu.VMEM((N,), jnp.int32),
            pltpu.VMEM((EPW,), jnp.int32),
            pltpu.VMEM((EPW,), jnp.int32),
            pltpu.VMEM((EPW,), jnp.int32),
            pltpu.VMEM((EPW,), jnp.int32),
        ],
        compiler_params=pltpu.CompilerParams(needs_layout_passes=False),
    )(_zgather_body)
    return k(z, row, col)


def _edge_body(r_ref, zr_ref, zc_ref, dW1_ref, db1_ref, Wd_ref, S2_ref,
               T2_ref, cvec_ref, eW2_ref, eb2_ref, out_ref):
    bf16 = jnp.bfloat16
    rb = r_ref[0]
    zr = zr_ref[0]
    zc = zc_ref[0]
    theta = (jnp.pi / CUTOFF) * rb
    c2 = 2.0 * jnp.cos(theta)
    t1 = jnp.sin(theta) * (jnp.sqrt(2.0 / CUTOFF) / rb)
    rows = [t1]
    tp2, tp1 = jnp.zeros_like(t1), t1
    for _ in range(NB - 1):
        t = c2 * tp1 - tp2
        tp2, tp1 = tp1, t
        rows.append(t)
    rows.extend([jnp.zeros_like(t1)] * (NBP - NB))
    basis = jnp.concatenate(rows, axis=0)
    tdot = (((0,), (0,)), ((), ()))
    hid = _silu_t(lax.dot_general(basis.astype(bf16), dW1_ref[...], tdot,
                                  preferred_element_type=jnp.float32)
                  + db1_ref[...])
    lanes = lax.broadcasted_iota(jnp.int32, (NUM_EL, BE), 0)
    ohr = (lanes == zr).astype(bf16)
    ohc = (lanes == zc).astype(bf16)
    pre = (jnp.dot(hid.astype(bf16), Wd_ref[...],
                   preferred_element_type=jnp.float32)
           + lax.dot_general(ohr, S2_ref[...], tdot,
                             preferred_element_type=jnp.float32)
           + lax.dot_general(ohc, T2_ref[...], tdot,
                             preferred_element_type=jnp.float32)
           + cvec_ref[...])
    out_ref[...] = (jnp.dot(_silu_t(pre).astype(bf16), eW2_ref[...],
                            preferred_element_type=jnp.float32)
                    + eb2_ref[...])


def _edge_feats(r3, zr3, zc3, dW1p, db1, Wd, S2, T2, cvec, eW2, eb2):
    full = lambda shape: pl.BlockSpec(shape, lambda i: tuple(0 for _ in shape))
    return pl.pallas_call(
        _edge_body,
        grid=(NBLK,),
        in_specs=[
            pl.BlockSpec((1, 1, BE), lambda i: (i, 0, 0)),
            pl.BlockSpec((1, 1, BE), lambda i: (i, 0, 0)),
            pl.BlockSpec((1, 1, BE), lambda i: (i, 0, 0)),
            full((NBP, F)), full((1, F)), full((F, F)), full((NUM_EL, F)),
            full((NUM_EL, F)), full((1, F)), full((F, F)), full((1, F)),
        ],
        out_specs=pl.BlockSpec((BE, F), lambda i: (i, 0)),
        out_shape=jax.ShapeDtypeStruct((E, F), jnp.float32),
    )(r3, zr3, zc3, dW1p, db1, Wd, S2, T2, cvec, eW2, eb2)


def _fill_iota(idx_ref, off):
    for k in range(STG // 16):
        idx_ref[pl.ds(k * 16, 16)] = lax.iota(jnp.int32, 16) + (off + k * 16)


def _scatter_body(ef_hbm, row_hbm, zacc_hbm, acc_hbm,
                  idx_v, rows_v, accst_v, iidx_v, tidx_v, trows_v,
                  acc_sh, *sems):
    lsems = sems[:RING]
    asems = sems[RING:]
    c = lax.axis_index("c")
    s = lax.axis_index("s")
    wid = s * NC + c
    base = wid * EPWS
    pltpu.sync_copy(zacc_hbm, accst_v)

    def initb(j, carry):
        _fill_iota(iidx_v, s * NPSA + j * STG)
        pltpu.sync_copy(accst_v, acc_sh.at[iidx_v])
        return carry

    lax.fori_loop(0, NSTG, initb, 0)

    @pl.when(s == NS - 1)
    def _():
        _fill_iota(iidx_v, N - STG)
        pltpu.sync_copy(accst_v, acc_sh.at[iidx_v])

    plsc.subcore_barrier()

    def _issue_loads(b, g):
        off = base + g * CHUNK
        pltpu.async_copy(row_hbm.at[pl.ds(off, CHUNK)], idx_v.at[b],
                         lsems[b])
        pltpu.async_copy(ef_hbm.at[pl.ds(off, CHUNK)], rows_v.at[b],
                         lsems[b])

    def _wait_loads(b, g):
        off = base + g * CHUNK
        pltpu.make_async_copy(row_hbm.at[pl.ds(off, CHUNK)], idx_v.at[b],
                              lsems[b]).wait()
        pltpu.make_async_copy(ef_hbm.at[pl.ds(off, CHUNK)], rows_v.at[b],
                              lsems[b]).wait()

    def _add_desc(b):
        return pltpu.make_async_copy(rows_v.at[b], acc_sh.at[idx_v.at[b]],
                                     asems[b])

    for b in range(RING):
        _issue_loads(b, b)

    def outer(o, carry):
        for b in range(RING):
            v = o * RING + b
            _wait_loads(b, v)
            _add_desc(b).start(add=True)
            bu = (b + 1) % RING
            u = v - (RING - 1)
            if b == RING - 1:
                @pl.when(o < NCHUNK // RING - 1)
                def _():
                    _add_desc(bu).wait()
                    _issue_loads(bu, u + RING)
            else:
                @pl.when(o > 0)
                def _():
                    _add_desc(bu).wait()
                    _issue_loads(bu, u + RING)
        return carry

    lax.fori_loop(0, NCHUNK // RING, outer, 0)
    for b in range(RING):
        _add_desc(b).wait()
    toff = base + NCHUNK * CHUNK
    pltpu.sync_copy(row_hbm.at[pl.ds(toff, TAILC)], tidx_v)
    pltpu.sync_copy(ef_hbm.at[pl.ds(toff, TAILC)], trows_v)
    pltpu.sync_copy(trows_v, acc_sh.at[tidx_v], add=True)
    plsc.subcore_barrier()

    def outb(j, carry):
        off = s * NPSA + j * STG
        _fill_iota(iidx_v, off)
        pltpu.sync_copy(acc_sh.at[iidx_v], accst_v)
        pltpu.sync_copy(accst_v, acc_hbm.at[pl.ds(c * N + off, STG)])
        return carry

    lax.fori_loop(0, NSTG, outb, 0)

    @pl.when(s == NS - 1)
    def _():
        _fill_iota(iidx_v, N - STG)
        pltpu.sync_copy(acc_sh.at[iidx_v], accst_v)
        pltpu.sync_copy(accst_v, acc_hbm.at[pl.ds(c * N + N - STG, STG)])


def _scatter(ef, row, zacc):
    mesh = plsc.VectorSubcoreMesh(core_axis_name="c", subcore_axis_name="s",
                                  num_cores=NC, num_subcores=NS)
    k = functools.partial(
        pl.kernel,
        mesh=mesh,
        out_type=jax.ShapeDtypeStruct((NC * N, F), jnp.float32),
        scratch_types=(
            [pltpu.VMEM((RING, CHUNK), jnp.int32),
             pltpu.VMEM((RING, CHUNK, F), jnp.float32),
             pltpu.VMEM((STG, F), jnp.float32),
             pltpu.VMEM((STG,), jnp.int32),
             pltpu.VMEM((TAILC,), jnp.int32),
             pltpu.VMEM((TAILC, F), jnp.float32),
             pltpu.VMEM_SHARED((N, F), jnp.float32)]
            + [pltpu.SemaphoreType.DMA] * (2 * RING)
        ),
    )(_scatter_body)
    return k(ef, row, zacc)


def _final_body(a_ref, b_ref, c_ref, d_ref, out_ref):
    out_ref[...] = (a_ref[...] + b_ref[...]) + (c_ref[...] + d_ref[...])


def _finalize(acc0, acc1):
    return pl.pallas_call(
        _final_body,
        grid=(NNBLK,),
        in_specs=[
            pl.BlockSpec((BN, F), lambda i: (i, 0)),
            pl.BlockSpec((BN, F), lambda i: (i + NNBLK, 0)),
            pl.BlockSpec((BN, F), lambda i: (i, 0)),
            pl.BlockSpec((BN, F), lambda i: (i + NNBLK, 0)),
        ],
        out_specs=pl.BlockSpec((BN, F), lambda i: (i, 0)),
        out_shape=jax.ShapeDtypeStruct((N, F), jnp.float32),
    )(acc0, acc0, acc1, acc1)


def kernel(z, edge_index, edge_weight, emb, dW1, db1, dW2, db2, sW1, sb1,
           sW2, sb2, tW1, tb1, tW2, tb2, eW1, eb1, eW2, eb2):
    f32 = jnp.float32
    row = edge_index[0].astype(jnp.int32)
    col = edge_index[1].astype(jnp.int32)
    z32 = z.astype(jnp.int32)

    S2, T2, Wd, cvec = _precompute(
        emb.astype(f32), sW1, sb1.reshape(1, F), sW2, sb2.reshape(1, F),
        tW1, tb1.reshape(1, F), tW2, tb2.reshape(1, F),
        dW2, db2.reshape(1, F), eW1, eb1.reshape(1, F))

    zr, zc = _zgather(z32, row, col)

    bf16 = jnp.bfloat16
    dW1p = jnp.pad(dW1, ((0, NBP - NB), (0, 0)))
    zacc = jnp.zeros((STG, F), f32)
    r_pieces = edge_weight.astype(f32).reshape(NPIECE, NBLK, 1, BE)
    zr_pieces = zr.reshape(NPIECE, NBLK, 1, BE)
    zc_pieces = zc.reshape(NPIECE, NBLK, 1, BE)
    row_pieces = row.reshape(NPIECE, EP)

    accs = []
    for p in range(NPIECE):
        ef = _edge_feats(
            r_pieces[p], zr_pieces[p], zc_pieces[p],
            dW1p.astype(bf16), db1.reshape(1, F), Wd.astype(bf16),
            S2.astype(bf16), T2.astype(bf16), cvec, eW2.astype(bf16),
            eb2.reshape(1, F))
        accs.append(_scatter(ef, row_pieces[p], zacc))

    return _finalize(accs[0], accs[1])


# speedup vs baseline: 1.6022x; 1.0021x over previous
import functools

import jax
import jax.numpy as jnp
from jax import lax
from jax.experimental import pallas as pl
from jax.experimental.pallas import tpu as pltpu
from jax.experimental.pallas import tpu_sc as plsc

N = 10000
E = 320000
NB = 20
NBP = 24
CUTOFF = 5.0
NUM_EL = 120
F = 128

NC = 2
NS = 16
NW = NC * NS
EPW = E // NW
NPIECE = 2
EP = E // NPIECE
EPWS = EP // NW
NPSA = 624
NTAIL = N - NS * NPSA
STG = 48
NSTG = NPSA // STG
CHUNK = 104
NCHUNK = EPWS // CHUNK
TAILC = EPWS - NCHUNK * CHUNK
RING = 3

BE = 16000
NBLK = EP // BE

BN = 1000
NNBLK = N // BN


def _sigmoid(x):
    return 1.0 / (1.0 + jnp.exp(-x))


def _silu(x):
    return x * _sigmoid(x)


def _silu_t(x):
    return 0.5 * x * (1.0 + jnp.tanh(0.5 * x))


def _precompute_body(emb_ref, sW1_ref, sb1_ref, sW2_ref, sb2_ref,
                     tW1_ref, tb1_ref, tW2_ref, tb2_ref,
                     dW2_ref, db2_ref, eW1_ref, eb1_ref,
                     S2_ref, T2_ref, Wd_ref, cvec_ref):
    emb = emb_ref[...]
    S = _silu(jnp.dot(emb, sW1_ref[...], preferred_element_type=jnp.float32)
              + sb1_ref[...])
    S = jnp.dot(S, sW2_ref[...], preferred_element_type=jnp.float32) + sb2_ref[...]
    T = _silu(jnp.dot(emb, tW1_ref[...], preferred_element_type=jnp.float32)
              + tb1_ref[...])
    T = jnp.dot(T, tW2_ref[...], preferred_element_type=jnp.float32) + tb2_ref[...]
    eW1_d = eW1_ref[0:128, :]
    eW1_s = eW1_ref[128:256, :]
    eW1_t = eW1_ref[256:384, :]
    S2_ref[...] = jnp.dot(S, eW1_s, preferred_element_type=jnp.float32)
    T2_ref[...] = jnp.dot(T, eW1_t, preferred_element_type=jnp.float32)
    Wd_ref[...] = jnp.dot(dW2_ref[...], eW1_d, preferred_element_type=jnp.float32)
    cvec_ref[...] = jnp.dot(db2_ref[...], eW1_d,
                            preferred_element_type=jnp.float32) + eb1_ref[...]


def _precompute(emb, sW1, sb1, sW2, sb2, tW1, tb1, tW2, tb2, dW2, db2, eW1, eb1):
    full = lambda shape: pl.BlockSpec(shape, lambda: tuple(0 for _ in shape))
    return pl.pallas_call(
        _precompute_body,
        grid=(),
        in_specs=[full((NUM_EL, F)), full((F, F)), full((1, F)), full((F, F)),
                  full((1, F)), full((F, F)), full((1, F)), full((F, F)),
                  full((1, F)), full((F, F)), full((1, F)), full((3 * F, F)),
                  full((1, F))],
        out_specs=[full((NUM_EL, F)), full((NUM_EL, F)), full((F, F)),
                   full((1, F))],
        out_shape=[jax.ShapeDtypeStruct((NUM_EL, F), jnp.float32),
                   jax.ShapeDtypeStruct((NUM_EL, F), jnp.float32),
                   jax.ShapeDtypeStruct((F, F), jnp.float32),
                   jax.ShapeDtypeStruct((1, F), jnp.float32)],
    )(emb, sW1, sb1, sW2, sb2, tW1, tb1, tW2, tb2, dW2, db2, eW1, eb1)


def _zgather_body(z_hbm, row_hbm, col_hbm, zr_hbm, zc_hbm,
                  z_v, row_v, col_v, zr_v, zc_v):
    c = lax.axis_index("c")
    s = lax.axis_index("s")
    wid = s * NC + c
    base = wid * EPW
    pltpu.sync_copy(z_hbm, z_v)
    pltpu.sync_copy(row_hbm.at[pl.ds(base, EPW)], row_v)
    pltpu.sync_copy(col_hbm.at[pl.ds(base, EPW)], col_v)

    def body(i, carry):
        off = i * 16
        idx_r = row_v[pl.ds(off, 16)]
        zr_v[pl.ds(off, 16)] = plsc.load_gather(z_v, [idx_r])
        idx_c = col_v[pl.ds(off, 16)]
        zc_v[pl.ds(off, 16)] = plsc.load_gather(z_v, [idx_c])
        return carry

    lax.fori_loop(0, EPW // 16, body, 0)
    pltpu.sync_copy(zr_v, zr_hbm.at[pl.ds(base, EPW)])
    pltpu.sync_copy(zc_v, zc_hbm.at[pl.ds(base, EPW)])


def _zgather(z, row, col):
    mesh = plsc.VectorSubcoreMesh(core_axis_name="c", subcore_axis_name="s",
                                  num_cores=NC, num_subcores=NS)
    k = functools.partial(
        pl.kernel,
        mesh=mesh,
        out_type=[jax.ShapeDtypeStruct((E,), jnp.int32),
                  jax.ShapeDtypeStruct((E,), jnp.int32)],
        scratch_types=[
            pltpu.VMEM((N,), jnp.int32),
            pltpu.VMEM((EPW,), jnp.int32),
            pltpu.VMEM((EPW,), jnp.int32),
            pltpu.VMEM((EPW,), jnp.int32),
            pltpu.VMEM((EPW,), jnp.int32),
        ],
        compiler_params=pltpu.CompilerParams(needs_layout_passes=False),
    )(_zgather_body)
    return k(z, row, col)


def _edge_body(r_ref, zr_ref, zc_ref, dW1_ref, db1_ref, Wd_ref, S2_ref,
               T2_ref, cvec_ref, eW2_ref, eb2_ref, out_ref):
    bf16 = jnp.bfloat16
    rb = r_ref[0]
    zr = zr_ref[0]
    zc = zc_ref[0]
    theta = (jnp.pi / CUTOFF) * rb
    c2 = 2.0 * jnp.cos(theta)
    t1 = jnp.sin(theta) * (jnp.sqrt(2.0 / CUTOFF) / rb)
    rows = [t1]
    tp2, tp1 = jnp.zeros_like(t1), t1
    for _ in range(NB - 1):
        t = c2 * tp1 - tp2
        tp2, tp1 = tp1, t
        rows.append(t)
    rows.extend([jnp.zeros_like(t1)] * (NBP - NB))
    basis = jnp.concatenate(rows, axis=0)
    tdot = (((0,), (0,)), ((), ()))
    hid = _silu_t(lax.dot_general(basis.astype(bf16), dW1_ref[...], tdot,
                                  preferred_element_type=jnp.float32)
                  + db1_ref[...])
    lanes = lax.broadcasted_iota(jnp.int32, (NUM_EL, BE), 0)
    ohr = (lanes == zr).astype(bf16)
    ohc = (lanes == zc).astype(bf16)
    pre = (jnp.dot(hid.astype(bf16), Wd_ref[...],
                   preferred_element_type=jnp.float32)
           + lax.dot_general(ohr, S2_ref[...], tdot,
                             preferred_element_type=jnp.float32)
           + lax.dot_general(ohc, T2_ref[...], tdot,
                             preferred_element_type=jnp.float32)
           + cvec_ref[...])
    out_ref[...] = (jnp.dot(_silu_t(pre).astype(bf16), eW2_ref[...],
                            preferred_element_type=jnp.float32)
                    + eb2_ref[...])


def _edge_feats(r3, zr3, zc3, dW1p, db1, Wd, S2, T2, cvec, eW2, eb2):
    full = lambda shape: pl.BlockSpec(shape, lambda i: tuple(0 for _ in shape))
    return pl.pallas_call(
        _edge_body,
        grid=(NBLK,),
        in_specs=[
            pl.BlockSpec((1, 1, BE), lambda i: (i, 0, 0)),
            pl.BlockSpec((1, 1, BE), lambda i: (i, 0, 0)),
            pl.BlockSpec((1, 1, BE), lambda i: (i, 0, 0)),
            full((NBP, F)), full((1, F)), full((F, F)), full((NUM_EL, F)),
            full((NUM_EL, F)), full((1, F)), full((F, F)), full((1, F)),
        ],
        out_specs=pl.BlockSpec((BE, F), lambda i: (i, 0)),
        out_shape=jax.ShapeDtypeStruct((E, F), jnp.float32),
    )(r3, zr3, zc3, dW1p, db1, Wd, S2, T2, cvec, eW2, eb2)


def _fill_iota(idx_ref, off):
    for k in range(STG // 16):
        idx_ref[pl.ds(k * 16, 16)] = lax.iota(jnp.int32, 16) + (off + k * 16)


def _scatter_body(ef_hbm, row_hbm, zacc_hbm, acc_hbm,
                  idx_v, rows_v, accst_v, iidx_v, tidx_v, trows_v,
                  acc_sh, *sems):
    lsems = sems[:RING]
    asems = sems[RING:]
    c = lax.axis_index("c")
    s = lax.axis_index("s")
    wid = s * NC + c
    base = wid * EPWS
    pltpu.sync_copy(zacc_hbm, accst_v)

    def initb(j, carry):
        _fill_iota(iidx_v, s * NPSA + j * STG)
        pltpu.sync_copy(accst_v, acc_sh.at[iidx_v])
        return carry

    lax.fori_loop(0, NSTG, initb, 0)

    @pl.when(s == NS - 1)
    def _():
        _fill_iota(iidx_v, N - STG)
        pltpu.sync_copy(accst_v, acc_sh.at[iidx_v])

    plsc.subcore_barrier()

    def _issue_loads(b, g):
        off = base + g * CHUNK
        pltpu.async_copy(row_hbm.at[pl.ds(off, CHUNK)], idx_v.at[b],
                         lsems[b])
        pltpu.async_copy(ef_hbm.at[pl.ds(off, CHUNK)], rows_v.at[b],
                         lsems[b])

    def _wait_loads(b, g):
        off = base + g * CHUNK
        pltpu.make_async_copy(row_hbm.at[pl.ds(off, CHUNK)], idx_v.at[b],
                              lsems[b]).wait()
        pltpu.make_async_copy(ef_hbm.at[pl.ds(off, CHUNK)], rows_v.at[b],
                              lsems[b]).wait()

    def _add_desc(b):
        return pltpu.make_async_copy(rows_v.at[b], acc_sh.at[idx_v.at[b]],
                                     asems[b])

    for b in range(RING):
        _issue_loads(b, b)

    def outer(o, carry):
        for b in range(RING):
            v = o * RING + b
            _wait_loads(b, v)
            _add_desc(b).start(add=True)
            bu = (b + 1) % RING
            u = v - (RING - 1)
            if b == RING - 1:
                @pl.when(o < NCHUNK // RING - 1)
                def _():
                    _add_desc(bu).wait()
                    _issue_loads(bu, u + RING)
            else:
                @pl.when(o > 0)
                def _():
                    _add_desc(bu).wait()
                    _issue_loads(bu, u + RING)
        return carry

    lax.fori_loop(0, NCHUNK // RING, outer, 0)
    for b in range(RING):
        _add_desc(b).wait()
    toff = base + NCHUNK * CHUNK
    pltpu.sync_copy(row_hbm.at[pl.ds(toff, TAILC)], tidx_v)
    pltpu.sync_copy(ef_hbm.at[pl.ds(toff, TAILC)], trows_v)
    pltpu.sync_copy(trows_v, acc_sh.at[tidx_v], add=True)
    plsc.subcore_barrier()

    def outb(j, carry):
        off = s * NPSA + j * STG
        _fill_iota(iidx_v, off)
        pltpu.sync_copy(acc_sh.at[iidx_v], accst_v)
        pltpu.sync_copy(accst_v, acc_hbm.at[pl.ds(c * N + off, STG)])
        return carry

    lax.fori_loop(0, NSTG, outb, 0)

    @pl.when(s == NS - 1)
    def _():
        _fill_iota(iidx_v, N - STG)
        pltpu.sync_copy(acc_sh.at[iidx_v], accst_v)
        pltpu.sync_copy(accst_v, acc_hbm.at[pl.ds(c * N + N - STG, STG)])


def _scatter(ef, row, zacc):
    mesh = plsc.VectorSubcoreMesh(core_axis_name="c", subcore_axis_name="s",
                                  num_cores=NC, num_subcores=NS)
    k = functools.partial(
        pl.kernel,
        mesh=mesh,
        out_type=jax.ShapeDtypeStruct((NC * N, F), jnp.float32),
        scratch_types=(
            [pltpu.VMEM((RING, CHUNK), jnp.int32),
             pltpu.VMEM((RING, CHUNK, F), jnp.float32),
             pltpu.VMEM((STG, F), jnp.float32),
             pltpu.VMEM((STG,), jnp.int32),
             pltpu.VMEM((TAILC,), jnp.int32),
             pltpu.VMEM((TAILC, F), jnp.float32),
             pltpu.VMEM_SHARED((N, F), jnp.float32)]
            + [pltpu.SemaphoreType.DMA] * (2 * RING)
        ),
    )(_scatter_body)
    return k(ef, row, zacc)


def _final_body(a_ref, b_ref, c_ref, d_ref, out_ref):
    out_ref[...] = (a_ref[...] + b_ref[...]) + (c_ref[...] + d_ref[...])


def _finalize(acc0, acc1):
    return pl.pallas_call(
        _final_body,
        grid=(NNBLK,),
        in_specs=[
            pl.BlockSpec((BN, F), lambda i: (i, 0)),
            pl.BlockSpec((BN, F), lambda i: (i + NNBLK, 0)),
            pl.BlockSpec((BN, F), lambda i: (i, 0)),
            pl.BlockSpec((BN, F), lambda i: (i + NNBLK, 0)),
        ],
        out_specs=pl.BlockSpec((BN, F), lambda i: (i, 0)),
        out_shape=jax.ShapeDtypeStruct((N, F), jnp.float32),
    )(acc0, acc0, acc1, acc1)


def kernel(z, edge_index, edge_weight, emb, dW1, db1, dW2, db2, sW1, sb1,
           sW2, sb2, tW1, tb1, tW2, tb2, eW1, eb1, eW2, eb2):
    f32 = jnp.float32
    row = edge_index[0].astype(jnp.int32)
    col = edge_index[1].astype(jnp.int32)
    z32 = z.astype(jnp.int32)

    S2, T2, Wd, cvec = _precompute(
        emb.astype(f32), sW1, sb1.reshape(1, F), sW2, sb2.reshape(1, F),
        tW1, tb1.reshape(1, F), tW2, tb2.reshape(1, F),
        dW2, db2.reshape(1, F), eW1, eb1.reshape(1, F))

    zr, zc = _zgather(z32, row, col)

    bf16 = jnp.bfloat16
    dW1p = jnp.pad(dW1, ((0, NBP - NB), (0, 0)))
    zacc = jnp.zeros((STG, F), f32)
    r_pieces = edge_weight.astype(f32).reshape(NPIECE, NBLK, 1, BE)
    zr_pieces = zr.reshape(NPIECE, NBLK, 1, BE)
    zc_pieces = zc.reshape(NPIECE, NBLK, 1, BE)
    row_pieces = row.reshape(NPIECE, EP)

    accs = []
    for p in range(NPIECE):
        ef = _edge_feats(
            r_pieces[p], zr_pieces[p], zc_pieces[p],
            dW1p.astype(bf16), db1.reshape(1, F), Wd.astype(bf16),
            S2.astype(bf16), T2.astype(bf16), cvec, eW2.astype(bf16),
            eb2.reshape(1, F))
        accs.append(_scatter(ef, row_pieces[p], zacc))

    return _finalize(accs[0], accs[1])
